# baseline (speedup 1.0000x reference)
import jax
import jax.numpy as jnp
from jax.experimental import pallas as pl


def _tc_body(x_ref, w_ref, o_ref):
    inv = 1.0 / x_ref.shape[1]
    s = jnp.sum(x_ref[0], axis=0, keepdims=True)
    o_ref[0] = jax.lax.dot_general(
        s, w_ref[...],
        dimension_numbers=(((1,), (1,)), ((), ())),
        preferred_element_type=jnp.float32,
    ) * inv


def kernel(x, W):
    B, C, H, Wsp = x.shape
    S = H * Wsp
    E = W.shape[0]
    xv = jnp.transpose(x, (0, 2, 3, 1)).reshape(B, S, C)
    out3 = pl.pallas_call(
        _tc_body,
        grid=(B,),
        in_specs=[
            pl.BlockSpec((1, S, C), lambda i: (i, 0, 0)),
            pl.BlockSpec((E, C), lambda i: (0, 0)),
        ],
        out_specs=pl.BlockSpec((1, 1, E), lambda i: (i, 0, 0)),
        out_shape=jax.ShapeDtypeStruct((B, 1, E), jnp.float32),
    )(xv, W)
    return out3.reshape(B, E)


# speedup vs baseline: 1.7939x; 1.7939x over previous
import jax
import jax.numpy as jnp
from jax.experimental import pallas as pl


def _tc_body(x_ref, w_ref, o_ref):
    inv = 1.0 / x_ref.shape[1]
    s = jnp.sum(x_ref[...], axis=1)
    o_ref[...] = jax.lax.dot_general(
        s, w_ref[...],
        dimension_numbers=(((1,), (1,)), ((), ())),
        preferred_element_type=jnp.float32,
    ) * inv


def kernel(x, W):
    B, C, H, Wsp = x.shape
    S = H * Wsp
    E = W.shape[0]
    xv = jnp.transpose(x, (0, 2, 3, 1)).reshape(B, S, C)
    Bblk = 16
    return pl.pallas_call(
        _tc_body,
        grid=(B // Bblk,),
        in_specs=[
            pl.BlockSpec((Bblk, S, C), lambda i: (i, 0, 0)),
            pl.BlockSpec((E, C), lambda i: (0, 0)),
        ],
        out_specs=pl.BlockSpec((Bblk, E), lambda i: (i, 0)),
        out_shape=jax.ShapeDtypeStruct((B, E), jnp.float32),
    )(xv, W)
